# Initial kernel scaffold; baseline (speedup 1.0000x reference)
import jax
import jax.numpy as jnp
from jax.experimental import pallas as pl


def kernel(edge_data, pred_leaf, X, leaf_emb, W1, b1, W2, b2, W3, b3, W4, b4):
    raise NotImplementedError("write your pallas kernel here")


# speedup vs baseline: 37.4155x; 37.4155x over previous
import functools

import jax
import jax.numpy as jnp
from jax import lax
from jax.experimental import pallas as pl
from jax.experimental.pallas import tpu as pltpu
from jax.experimental.pallas import tpu_sc as plsc

_NC = 2
_NS = 16
_NW = _NC * _NS
_BLK = 512


def _sc_gather_rows(x, idx):
    d = x.shape[1]
    per = idx.shape[0] // _NW
    idx2 = idx.reshape(_NW, per)
    mesh = plsc.VectorSubcoreMesh(core_axis_name="c", subcore_axis_name="s")

    @functools.partial(
        pl.kernel,
        mesh=mesh,
        out_type=jax.ShapeDtypeStruct((_NW, per, d), jnp.float32),
        scratch_types=[
            pltpu.VMEM((per,), jnp.int32),
            pltpu.VMEM((per // 2, d), jnp.float32),
            pltpu.SemaphoreType.DMA,
        ],
    )
    def gather_kernel(x_hbm, idx_hbm, out_hbm, idx_v, rows_v, sem):
        wid = lax.axis_index("s") * _NC + lax.axis_index("c")
        pltpu.sync_copy(idx_hbm.at[wid], idx_v)
        half = per // 2

        for rnd in range(2):
            def start(gg, carry):
                vec = idx_v[pl.ds(rnd * half + gg * 16, 16)]
                for j in range(16):
                    pltpu.make_async_copy(
                        x_hbm.at[pl.ds(vec[j], 1)],
                        rows_v.at[pl.ds(gg * 16 + j, 1)],
                        sem).start()
                return carry

            lax.fori_loop(0, half // 16, start, 0)

            def drain(i, carry):
                pltpu.make_async_copy(
                    x_hbm.at[pl.ds(0, 1)], rows_v.at[pl.ds(0, 1)],
                    sem).wait()
                return carry

            lax.fori_loop(0, half, drain, 0)
            pltpu.sync_copy(rows_v, out_hbm.at[wid, pl.ds(rnd * half, half)])

    return gather_kernel(x, idx2).reshape(_NW * per, d)


def _mlp_body(pl_ref, gc_ref, gd_ref, e_ref, tp_ref, w1c_ref, w1d_ref,
              b1_ref, w2_ref, b2_ref, w3_ref, b3_ref, w4_ref, b4_ref,
              out_ref):
    pf = pl_ref[...].astype(jnp.bfloat16)
    rep = jnp.dot(pf, e_ref[...], preferred_element_type=jnp.float32)
    band = (lax.broadcasted_iota(jnp.int32, rep.shape, 1) & 31)
    oh = jnp.where(rep == band.astype(jnp.float32),
                   jnp.float32(1.0), jnp.float32(0.0))
    h = jnp.dot(oh, tp_ref[...], preferred_element_type=jnp.float32)
    h = h + jnp.dot(gc_ref[...], w1c_ref[...],
                    preferred_element_type=jnp.float32)
    h = h + jnp.dot(gd_ref[...], w1d_ref[...],
                    preferred_element_type=jnp.float32)
    h = jnp.maximum(h + b1_ref[...], 0.0)
    h = jnp.maximum(
        jnp.dot(h, w2_ref[...], preferred_element_type=jnp.float32)
        + b2_ref[...], 0.0)
    h = jnp.maximum(
        jnp.dot(h, w3_ref[...], preferred_element_type=jnp.float32)
        + b3_ref[...], 0.0)
    z = (jnp.dot(h, w4_ref[...], preferred_element_type=jnp.float32)
         + b4_ref[...])
    out_ref[...] = jax.nn.sigmoid(z)


def kernel(edge_data, pred_leaf, X, leaf_emb, W1, b1, W2, b2, W3, b3, W4, b4):
    B, T = pred_leaf.shape
    L = leaf_emb.shape[1]
    Lp = 32
    D = leaf_emb.shape[2]
    F = X.shape[1]

    chan = edge_data[:, 0]
    dev = edge_data[:, 1]
    labels = edge_data[:, 2]

    idx = jnp.concatenate([chan, dev], axis=0)
    rows = _sc_gather_rows(X, idx)

    table = jnp.pad(leaf_emb, ((0, 0), (0, Lp - L), (0, 0))).reshape(T * Lp, D)
    tp = (table / T) @ W1[42:]
    w1c = jnp.pad(W1[:20], ((0, F - 20), (0, 0)))
    w1d = W1[20:42]
    e = (lax.broadcasted_iota(jnp.int32, (T, T * Lp), 1) // Lp
         == lax.broadcasted_iota(jnp.int32, (T, T * Lp), 0)
         ).astype(jnp.bfloat16)

    nblk = B // _BLK
    full = lambda i: (0, 0)
    blk = lambda i: (i, 0)
    blk2 = lambda i: (i + nblk, 0)
    h = pl.pallas_call(
        _mlp_body,
        grid=(nblk,),
        in_specs=[
            pl.BlockSpec((_BLK, T), blk),
            pl.BlockSpec((_BLK, F), blk),
            pl.BlockSpec((_BLK, F), blk2),
            pl.BlockSpec((T, T * Lp), full),
            pl.BlockSpec((T * Lp, 36), full),
            pl.BlockSpec((F, 36), full),
            pl.BlockSpec((F, 36), full),
            pl.BlockSpec((1, 36), full),
            pl.BlockSpec((36, 20), full),
            pl.BlockSpec((1, 20), full),
            pl.BlockSpec((20, 12), full),
            pl.BlockSpec((1, 12), full),
            pl.BlockSpec((12, 1), full),
            pl.BlockSpec((1, 1), full),
        ],
        out_specs=pl.BlockSpec((_BLK, 1), blk),
        out_shape=jax.ShapeDtypeStruct((B, 1), jnp.float32),
        compiler_params=pltpu.CompilerParams(
            dimension_semantics=("arbitrary",)),
    )(pred_leaf, rows, rows, e, tp, w1c, w1d, b1.reshape(1, 36),
      W2, b2.reshape(1, 20), W3, b3.reshape(1, 12), W4, b4.reshape(1, 1))
    return (h, labels)
